# baseline (speedup 1.0000x reference)
import functools

import jax
import jax.numpy as jnp
from jax import lax
from jax.experimental import pallas as pl
from jax.experimental.pallas import tpu as pltpu
from jax.experimental.pallas import tpu_sc as plsc

N = 10000
E = 320000
D = 128
H = 128

NC = 2
NS = 16
NW = NC * NS
B = 40
NB_T = E // (NS * B)
RING = 5
NGRP = NB_T // RING
CH = 40
NCHUNK = N // CH
BASE_NCH = NCHUNK // NS
EXTRA_CH = NCHUNK - BASE_NCH * NS

_MESH = plsc.VectorSubcoreMesh(
    core_axis_name="c", subcore_axis_name="s", num_cores=NC, num_subcores=NS)


@functools.partial(
    pl.kernel,
    mesh=_MESH,
    out_type=(
        jax.ShapeDtypeStruct((N, H), jnp.float32),
        jax.ShapeDtypeStruct((N, H), jnp.float32),
    ),
    scratch_types=(
        pltpu.VMEM((RING, B), jnp.int32),
        pltpu.VMEM((RING, B), jnp.int32),
        [pltpu.VMEM((B, H), jnp.float32)] * RING,
        pltpu.VMEM((CH, H), jnp.float32),
        pltpu.VMEM_SHARED((N, H), jnp.float32),
        [pltpu.SemaphoreType.DMA] * RING,
        [pltpu.SemaphoreType.DMA] * RING,
        pltpu.SemaphoreType.DMA,
        pltpu.SemaphoreType.DMA,
    ),
)
def _sc_segment_sums(src_hbm, dst_hbm, h_hbm, c_hbm, hout, cout,
                     sidx, didx, rows, zbuf, acc, gsem, ssem, isem_s, isem_d):
    c_id = lax.axis_index("c")
    s_id = lax.axis_index("s")

    zeros16 = jnp.zeros((16,), jnp.float32)

    def zfill(r, carry):
        for k in range(H // 16):
            zbuf[r, pl.ds(k * 16, 16)] = zeros16
        return carry
    lax.fori_loop(0, CH, zfill, 0)

    nch = BASE_NCH + jnp.where(s_id < EXTRA_CH, 1, 0)

    def zero_chunks():
        def body(k, carry):
            r0 = (s_id + NS * k) * CH
            pltpu.sync_copy(zbuf, acc.at[pl.ds(r0, CH), :])
            return carry
        lax.fori_loop(0, nch, body, 0)

    def write_chunks(out_hbm):
        def body(k, carry):
            r0 = (s_id + NS * k) * CH
            pltpu.sync_copy(acc.at[pl.ds(r0, CH), :],
                            out_hbm.at[pl.ds(r0, CH), :])
            return carry
        lax.fori_loop(0, nch, body, 0)

    base = s_id * NB_T * B

    def fire_idx(j, b):
        pltpu.async_copy(src_hbm.at[pl.ds(base + j * B, B)], sidx.at[b],
                         isem_s)
        pltpu.async_copy(dst_hbm.at[pl.ds(base + j * B, B)], didx.at[b],
                         isem_d)

    def wait_idx(j, b):
        pltpu.make_async_copy(src_hbm.at[pl.ds(base + j * B, B)], sidx.at[b],
                              isem_s).wait()
        pltpu.make_async_copy(dst_hbm.at[pl.ds(base + j * B, B)], didx.at[b],
                              isem_d).wait()

    def edge_pass(table_hbm):
        def fire_gather(b):
            pltpu.async_copy(table_hbm.at[sidx.at[b]], rows[b], gsem[b])

        def wait_gather(b):
            pltpu.make_async_copy(table_hbm.at[sidx.at[b]], rows[b],
                                  gsem[b]).wait()

        def fire_scatter(b):
            pltpu.async_copy(rows[b], acc.at[didx.at[b]], ssem[b],
                             add=True)

        def wait_scatter(b):
            pltpu.make_async_copy(rows[b], acc.at[didx.at[b]],
                                  ssem[b]).wait()

        for b in range(RING):
            fire_idx(b, b)
        for b in range(RING):
            wait_idx(b, b)
            fire_gather(b)

        def body(g, carry):
            j0 = g * RING
            for b in range(RING):
                wait_gather(b)
                fire_scatter(b)
            for b in range(RING):
                wait_scatter(b)
                fire_idx(j0 + RING + b, b)
            for b in range(RING):
                wait_idx(j0 + RING + b, b)
                fire_gather(b)
            return carry
        lax.fori_loop(0, NGRP - 1, body, 0)

        for b in range(RING):
            wait_gather(b)
            fire_scatter(b)
        for b in range(RING):
            wait_scatter(b)

    zero_chunks()
    plsc.subcore_barrier()

    @pl.when(c_id == 0)
    def _h_sweep():
        edge_pass(h_hbm)

    @pl.when(c_id == 1)
    def _c_sweep():
        edge_pass(c_hbm)

    plsc.subcore_barrier()

    @pl.when(c_id == 0)
    def _h_write():
        write_chunks(hout)

    @pl.when(c_id == 1)
    def _c_write():
        write_chunks(cout)


BLK = 2000


def _tc_body(x_ref, hp_ref, cp_ref, wf_ref, uf_ref, bf_ref,
             wiou_ref, uiou_ref, biou_ref, hnew_ref, cnew_ref):
    xb = x_ref[...]
    h_sum = hp_ref[...]
    c_sum = cp_ref[...]
    f = jax.nn.sigmoid(
        jnp.dot(xb, wf_ref[...], preferred_element_type=jnp.float32)
        + jnp.dot(h_sum, uf_ref[...], preferred_element_type=jnp.float32)
        + bf_ref[...])
    iou = (jnp.dot(xb, wiou_ref[...], preferred_element_type=jnp.float32)
           + jnp.dot(h_sum, uiou_ref[...], preferred_element_type=jnp.float32)
           + biou_ref[...])
    i = jax.nn.sigmoid(iou[:, :H])
    o = jax.nn.sigmoid(iou[:, H:2 * H])
    u = jnp.tanh(iou[:, 2 * H:])
    c_new = i * u + f * c_sum
    cnew_ref[...] = c_new
    hnew_ref[...] = o * jnp.tanh(c_new)


def _tc_dense(x, hsum_parts, csum_parts, wf_t, uf_t, b_f, wiou_t, uiou_t, b_iou):
    grid = (N // BLK,)
    return pl.pallas_call(
        _tc_body,
        grid=grid,
        in_specs=[
            pl.BlockSpec((BLK, D), lambda i: (i, 0)),
            pl.BlockSpec((BLK, H), lambda i: (i, 0)),
            pl.BlockSpec((BLK, H), lambda i: (i, 0)),
            pl.BlockSpec((D, H), lambda i: (0, 0)),
            pl.BlockSpec((H, H), lambda i: (0, 0)),
            pl.BlockSpec((1, H), lambda i: (0, 0)),
            pl.BlockSpec((D, 3 * H), lambda i: (0, 0)),
            pl.BlockSpec((H, 3 * H), lambda i: (0, 0)),
            pl.BlockSpec((1, 3 * H), lambda i: (0, 0)),
        ],
        out_specs=[
            pl.BlockSpec((BLK, H), lambda i: (i, 0)),
            pl.BlockSpec((BLK, H), lambda i: (i, 0)),
        ],
        out_shape=[
            jax.ShapeDtypeStruct((N, H), jnp.float32),
            jax.ShapeDtypeStruct((N, H), jnp.float32),
        ],
    )(x, hsum_parts, csum_parts, wf_t, uf_t, b_f, wiou_t, uiou_t, b_iou)


def kernel(x, h, c, edge_index, W_f, U_f, b_f, W_iou, U_iou, b_iou):
    src = edge_index[0].astype(jnp.int32)
    dst = edge_index[1].astype(jnp.int32)
    hsum_parts, csum_parts = _sc_segment_sums(src, dst, h, c)
    h_new, c_new = _tc_dense(x, hsum_parts, csum_parts,
                             W_f.T, U_f.T, b_f, W_iou.T, U_iou.T, b_iou)
    return h_new, c_new


# speedup vs baseline: 1.3112x; 1.3112x over previous
import functools

import jax
import jax.numpy as jnp
from jax import lax
from jax.experimental import pallas as pl
from jax.experimental.pallas import tpu as pltpu
from jax.experimental.pallas import tpu_sc as plsc

N = 10000
E = 320000
D = 128
H = 128

NC = 2
NS = 16
NW = NC * NS
B = 50
NB_T = E // (NS * B)
RING = 5
NGRP = NB_T // RING
CH = 40
NCHUNK = N // CH
BASE_NCH = NCHUNK // NS
EXTRA_CH = NCHUNK - BASE_NCH * NS

_MESH = plsc.VectorSubcoreMesh(
    core_axis_name="c", subcore_axis_name="s", num_cores=NC, num_subcores=NS)


@functools.partial(
    pl.kernel,
    mesh=_MESH,
    out_type=(
        jax.ShapeDtypeStruct((N, H), jnp.float32),
        jax.ShapeDtypeStruct((N, H), jnp.float32),
    ),
    scratch_types=(
        pltpu.VMEM((2, RING, B), jnp.int32),
        pltpu.VMEM((2, RING, B), jnp.int32),
        [pltpu.VMEM((B, H), jnp.float32)] * RING,
        pltpu.VMEM((CH, H), jnp.float32),
        pltpu.VMEM_SHARED((N, H), jnp.float32),
        [pltpu.SemaphoreType.DMA] * RING,
        [pltpu.SemaphoreType.DMA] * RING,
        pltpu.SemaphoreType.DMA,
        pltpu.SemaphoreType.DMA,
    ),
)
def _sc_segment_sums(src_hbm, dst_hbm, h_hbm, c_hbm, hout, cout,
                     sidx, didx, rows, zbuf, acc, gsem, ssem, isem_s, isem_d):
    c_id = lax.axis_index("c")
    s_id = lax.axis_index("s")

    zeros16 = jnp.zeros((16,), jnp.float32)

    def zfill(r, carry):
        for k in range(H // 16):
            zbuf[r, pl.ds(k * 16, 16)] = zeros16
        return carry
    lax.fori_loop(0, CH, zfill, 0)

    nch = BASE_NCH + jnp.where(s_id < EXTRA_CH, 1, 0)

    def zero_chunks():
        def body(k, carry):
            r0 = (s_id + NS * k) * CH
            pltpu.sync_copy(zbuf, acc.at[pl.ds(r0, CH), :])
            return carry
        lax.fori_loop(0, nch, body, 0)

    def write_chunks(out_hbm):
        def body(k, carry):
            r0 = (s_id + NS * k) * CH
            pltpu.sync_copy(acc.at[pl.ds(r0, CH), :],
                            out_hbm.at[pl.ds(r0, CH), :])
            return carry
        lax.fori_loop(0, nch, body, 0)

    def fire_idx(g, slot):
        pltpu.async_copy(src_hbm.at[s_id, g], sidx.at[slot], isem_s)
        pltpu.async_copy(dst_hbm.at[s_id, g], didx.at[slot], isem_d)

    def wait_idx(g, slot):
        pltpu.make_async_copy(src_hbm.at[s_id, g], sidx.at[slot],
                              isem_s).wait()
        pltpu.make_async_copy(dst_hbm.at[s_id, g], didx.at[slot],
                              isem_d).wait()

    def edge_pass(table_hbm):
        def fire_gather(slot, b):
            pltpu.async_copy(table_hbm.at[sidx.at[slot, b]], rows[b], gsem[b])

        def wait_gather(slot, b):
            pltpu.make_async_copy(table_hbm.at[sidx.at[slot, b]], rows[b],
                                  gsem[b]).wait()

        def fire_scatter(slot, b):
            pltpu.async_copy(rows[b], acc.at[didx.at[slot, b]], ssem[b],
                             add=True)

        def wait_scatter(slot, b):
            pltpu.make_async_copy(rows[b], acc.at[didx.at[slot, b]],
                                  ssem[b]).wait()

        fire_idx(0, 0)
        wait_idx(0, 0)
        for b in range(RING):
            fire_gather(0, b)

        def body(g, carry):
            slot = lax.rem(g, 2)
            nslot = lax.rem(g + 1, 2)
            fire_idx(g + 1, nslot)
            for b in range(RING):
                wait_gather(slot, b)
                fire_scatter(slot, b)
            wait_idx(g + 1, nslot)
            for b in range(RING):
                wait_scatter(slot, b)
                fire_gather(nslot, b)
            return carry
        lax.fori_loop(0, NGRP - 1, body, 0)

        last = (NGRP - 1) % 2
        for b in range(RING):
            wait_gather(last, b)
            fire_scatter(last, b)
        for b in range(RING):
            wait_scatter(last, b)

    zero_chunks()
    plsc.subcore_barrier()

    @pl.when(c_id == 0)
    def _h_sweep():
        edge_pass(h_hbm)

    @pl.when(c_id == 1)
    def _c_sweep():
        edge_pass(c_hbm)

    plsc.subcore_barrier()

    @pl.when(c_id == 0)
    def _h_write():
        write_chunks(hout)

    @pl.when(c_id == 1)
    def _c_write():
        write_chunks(cout)


BLK = 2000


def _tc_body(x_ref, hp_ref, cp_ref, wf_ref, uf_ref, bf_ref,
             wiou_ref, uiou_ref, biou_ref, hnew_ref, cnew_ref):
    xb = x_ref[...]
    h_sum = hp_ref[...]
    c_sum = cp_ref[...]
    f = jax.nn.sigmoid(
        jnp.dot(xb, wf_ref[...], preferred_element_type=jnp.float32)
        + jnp.dot(h_sum, uf_ref[...], preferred_element_type=jnp.float32)
        + bf_ref[...])
    iou = (jnp.dot(xb, wiou_ref[...], preferred_element_type=jnp.float32)
           + jnp.dot(h_sum, uiou_ref[...], preferred_element_type=jnp.float32)
           + biou_ref[...])
    i = jax.nn.sigmoid(iou[:, :H])
    o = jax.nn.sigmoid(iou[:, H:2 * H])
    u = jnp.tanh(iou[:, 2 * H:])
    c_new = i * u + f * c_sum
    cnew_ref[...] = c_new
    hnew_ref[...] = o * jnp.tanh(c_new)


def _tc_dense(x, hsum_parts, csum_parts, wf_t, uf_t, b_f, wiou_t, uiou_t, b_iou):
    grid = (N // BLK,)
    return pl.pallas_call(
        _tc_body,
        grid=grid,
        in_specs=[
            pl.BlockSpec((BLK, D), lambda i: (i, 0)),
            pl.BlockSpec((BLK, H), lambda i: (i, 0)),
            pl.BlockSpec((BLK, H), lambda i: (i, 0)),
            pl.BlockSpec((D, H), lambda i: (0, 0)),
            pl.BlockSpec((H, H), lambda i: (0, 0)),
            pl.BlockSpec((1, H), lambda i: (0, 0)),
            pl.BlockSpec((D, 3 * H), lambda i: (0, 0)),
            pl.BlockSpec((H, 3 * H), lambda i: (0, 0)),
            pl.BlockSpec((1, 3 * H), lambda i: (0, 0)),
        ],
        out_specs=[
            pl.BlockSpec((BLK, H), lambda i: (i, 0)),
            pl.BlockSpec((BLK, H), lambda i: (i, 0)),
        ],
        out_shape=[
            jax.ShapeDtypeStruct((N, H), jnp.float32),
            jax.ShapeDtypeStruct((N, H), jnp.float32),
        ],
    )(x, hsum_parts, csum_parts, wf_t, uf_t, b_f, wiou_t, uiou_t, b_iou)


def _repack_body(ei_ref, s_ref, d_ref):
    s_ref[...] = ei_ref[0].reshape(1, NGRP, RING, B)
    d_ref[...] = ei_ref[1].reshape(1, NGRP, RING, B)


def _repack(edge_index):
    per = NGRP * RING * B
    return pl.pallas_call(
        _repack_body,
        grid=(NS,),
        in_specs=[pl.BlockSpec((2, per), lambda i: (0, i))],
        out_specs=[
            pl.BlockSpec((1, NGRP, RING, B), lambda i: (i, 0, 0, 0)),
            pl.BlockSpec((1, NGRP, RING, B), lambda i: (i, 0, 0, 0)),
        ],
        out_shape=[
            jax.ShapeDtypeStruct((NS, NGRP, RING, B), jnp.int32),
            jax.ShapeDtypeStruct((NS, NGRP, RING, B), jnp.int32),
        ],
    )(edge_index)


def kernel(x, h, c, edge_index, W_f, U_f, b_f, W_iou, U_iou, b_iou):
    src = edge_index[0].astype(jnp.int32).reshape(NS, NGRP, RING, B)
    dst = edge_index[1].astype(jnp.int32).reshape(NS, NGRP, RING, B)
    hsum_parts, csum_parts = _sc_segment_sums(src, dst, h, c)
    h_new, c_new = _tc_dense(x, hsum_parts, csum_parts,
                             W_f.T, U_f.T, b_f, W_iou.T, U_iou.T, b_iou)
    return h_new, c_new


# speedup vs baseline: 1.3114x; 1.0002x over previous
import functools

import jax
import jax.numpy as jnp
from jax import lax
from jax.experimental import pallas as pl
from jax.experimental.pallas import tpu as pltpu
from jax.experimental.pallas import tpu_sc as plsc

N = 10000
E = 320000
D = 128
H = 128

NC = 2
NS = 16
NW = NC * NS
B = 50
NB_T = E // (NS * B)
RING = 5
NGRP = NB_T // RING
CH = 40
NCHUNK = N // CH
BASE_NCH = NCHUNK // NS
EXTRA_CH = NCHUNK - BASE_NCH * NS

_MESH = plsc.VectorSubcoreMesh(
    core_axis_name="c", subcore_axis_name="s", num_cores=NC, num_subcores=NS)


@functools.partial(
    pl.kernel,
    mesh=_MESH,
    out_type=(
        jax.ShapeDtypeStruct((N, H), jnp.float32),
        jax.ShapeDtypeStruct((N, H), jnp.float32),
    ),
    scratch_types=(
        pltpu.VMEM((2, RING, B), jnp.int32),
        pltpu.VMEM((2, RING, B), jnp.int32),
        [pltpu.VMEM((B, H), jnp.float32)] * RING,
        pltpu.VMEM((CH, H), jnp.float32),
        pltpu.VMEM_SHARED((N, H), jnp.float32),
        [pltpu.SemaphoreType.DMA] * RING,
        [pltpu.SemaphoreType.DMA] * RING,
        pltpu.SemaphoreType.DMA,
        pltpu.SemaphoreType.DMA,
    ),
)
def _sc_segment_sums(src_hbm, dst_hbm, h_hbm, c_hbm, hout, cout,
                     sidx, didx, rows, zbuf, acc, gsem, ssem, isem_s, isem_d):
    c_id = lax.axis_index("c")
    s_id = lax.axis_index("s")

    zeros16 = jnp.zeros((16,), jnp.float32)

    def zfill(r, carry):
        for k in range(H // 16):
            zbuf[r, pl.ds(k * 16, 16)] = zeros16
        return carry
    lax.fori_loop(0, CH, zfill, 0)

    nch = BASE_NCH + jnp.where(s_id < EXTRA_CH, 1, 0)

    def zero_chunks():
        def body(k, carry):
            r0 = (s_id + NS * k) * CH
            pltpu.sync_copy(zbuf, acc.at[pl.ds(r0, CH), :])
            return carry
        lax.fori_loop(0, nch, body, 0)

    def write_chunks(out_hbm):
        def body(k, carry):
            r0 = (s_id + NS * k) * CH
            pltpu.sync_copy(acc.at[pl.ds(r0, CH), :],
                            out_hbm.at[pl.ds(r0, CH), :])
            return carry
        lax.fori_loop(0, nch, body, 0)

    def fire_idx(g, slot):
        pltpu.async_copy(src_hbm.at[s_id, g], sidx.at[slot], isem_s)
        pltpu.async_copy(dst_hbm.at[s_id, g], didx.at[slot], isem_d)

    def wait_idx(g, slot):
        pltpu.make_async_copy(src_hbm.at[s_id, g], sidx.at[slot],
                              isem_s).wait()
        pltpu.make_async_copy(dst_hbm.at[s_id, g], didx.at[slot],
                              isem_d).wait()

    def edge_pass(table_hbm):
        def fire_gather(slot, b):
            pltpu.async_copy(table_hbm.at[sidx.at[slot, b]], rows[b], gsem[b])

        def wait_gather(slot, b):
            pltpu.make_async_copy(table_hbm.at[sidx.at[slot, b]], rows[b],
                                  gsem[b]).wait()

        def fire_scatter(slot, b):
            pltpu.async_copy(rows[b], acc.at[didx.at[slot, b]], ssem[b],
                             add=True)

        def wait_scatter(slot, b):
            pltpu.make_async_copy(rows[b], acc.at[didx.at[slot, b]],
                                  ssem[b]).wait()

        fire_idx(0, 0)
        wait_idx(0, 0)
        for b in range(RING):
            fire_gather(0, b)

        def body(g, carry):
            slot = lax.rem(g, 2)
            nslot = lax.rem(g + 1, 2)
            fire_idx(g + 1, nslot)
            for b in range(RING):
                wait_gather(slot, b)
                fire_scatter(slot, b)
            wait_idx(g + 1, nslot)
            for b in range(RING):
                wait_scatter(slot, b)
                fire_gather(nslot, b)
            return carry
        lax.fori_loop(0, NGRP - 1, body, 0)

        last = (NGRP - 1) % 2
        for b in range(RING):
            wait_gather(last, b)
            fire_scatter(last, b)
        for b in range(RING):
            wait_scatter(last, b)

    zero_chunks()
    plsc.subcore_barrier()

    @pl.when(c_id == 0)
    def _h_sweep():
        edge_pass(h_hbm)

    @pl.when(c_id == 1)
    def _c_sweep():
        edge_pass(c_hbm)

    plsc.subcore_barrier()

    @pl.when(c_id == 0)
    def _h_write():
        write_chunks(hout)

    @pl.when(c_id == 1)
    def _c_write():
        write_chunks(cout)


BLK = 2000


def _tc_body(x_ref, hp_ref, cp_ref, wf_ref, uf_ref, bf_ref,
             wiou_ref, uiou_ref, biou_ref, hnew_ref, cnew_ref):
    xb = x_ref[...]
    h_sum = hp_ref[...]
    c_sum = cp_ref[...]
    f = jax.nn.sigmoid(
        jnp.dot(xb, wf_ref[...], preferred_element_type=jnp.float32)
        + jnp.dot(h_sum, uf_ref[...], preferred_element_type=jnp.float32)
        + bf_ref[...])
    iou = (jnp.dot(xb, wiou_ref[...], preferred_element_type=jnp.float32)
           + jnp.dot(h_sum, uiou_ref[...], preferred_element_type=jnp.float32)
           + biou_ref[...])
    i = jax.nn.sigmoid(iou[:, :H])
    o = jax.nn.sigmoid(iou[:, H:2 * H])
    u = jnp.tanh(iou[:, 2 * H:])
    c_new = i * u + f * c_sum
    cnew_ref[...] = c_new
    hnew_ref[...] = o * jnp.tanh(c_new)


def _tc_dense(x, hsum_parts, csum_parts, wf_t, uf_t, b_f, wiou_t, uiou_t, b_iou):
    grid = (N // BLK,)
    return pl.pallas_call(
        _tc_body,
        grid=grid,
        in_specs=[
            pl.BlockSpec((BLK, D), lambda i: (i, 0)),
            pl.BlockSpec((BLK, H), lambda i: (i, 0)),
            pl.BlockSpec((BLK, H), lambda i: (i, 0)),
            pl.BlockSpec((D, H), lambda i: (0, 0)),
            pl.BlockSpec((H, H), lambda i: (0, 0)),
            pl.BlockSpec((1, H), lambda i: (0, 0)),
            pl.BlockSpec((D, 3 * H), lambda i: (0, 0)),
            pl.BlockSpec((H, 3 * H), lambda i: (0, 0)),
            pl.BlockSpec((1, 3 * H), lambda i: (0, 0)),
        ],
        out_specs=[
            pl.BlockSpec((BLK, H), lambda i: (i, 0)),
            pl.BlockSpec((BLK, H), lambda i: (i, 0)),
        ],
        out_shape=[
            jax.ShapeDtypeStruct((N, H), jnp.float32),
            jax.ShapeDtypeStruct((N, H), jnp.float32),
        ],
    )(x, hsum_parts, csum_parts, wf_t, uf_t, b_f, wiou_t, uiou_t, b_iou)


def kernel(x, h, c, edge_index, W_f, U_f, b_f, W_iou, U_iou, b_iou):
    src = edge_index[0].astype(jnp.int32).reshape(NS, NGRP, RING, B)
    dst = edge_index[1].astype(jnp.int32).reshape(NS, NGRP, RING, B)
    hsum_parts, csum_parts = _sc_segment_sums(src, dst, h, c)
    h_new, c_new = _tc_dense(x, hsum_parts, csum_parts,
                             W_f.T, U_f.T, b_f, W_iou.T, U_iou.T, b_iou)
    return h_new, c_new
